# baseline (speedup 1.0000x reference)
import functools
import math

import jax
import jax.numpy as jnp
from jax.experimental import pallas as pl
from jax.experimental.pallas import tpu as pltpu
from jax.experimental.pallas import tpu_sc as plsc

EMB_DIM = 32
SCALE = math.sqrt(float(EMB_DIM))
LANES = 16
N_L = 200
N_B = 4096
BB = 128


def _build_kernel():
    mesh = plsc.VectorSubcoreMesh(core_axis_name="c", subcore_axis_name="s")
    cp = pltpu.CompilerParams(
        needs_layout_passes=False, use_tc_tiling_on_sc=False
    )
    nbt = N_B // BB

    @functools.partial(
        pl.kernel,
        out_type=jax.ShapeDtypeStruct((N_L, 4, nbt, 8, BB), jnp.float32),
        mesh=mesh,
        compiler_params=cp,
        scratch_types=[
            pltpu.VMEM((BB, EMB_DIM), jnp.float32),
            pltpu.VMEM((BB,), jnp.int32),
        ],
    )
    def emb_kernel(table_hbm, idx_hbm, out_hbm, g_scr, i4_scr):
        iota16 = jax.lax.iota(jnp.int32, LANES)

        def body(i_vmem, o_vmem):
            for bc in range(BB // LANES):
                t16 = i_vmem[0, pl.ds(LANES * bc, LANES)]
                i4_scr[pl.ds(LANES * bc, LANES)] = t16 * 4
            pltpu.sync_copy(table_hbm.at[i4_scr], g_scr)
            for bc in range(BB // LANES):
                t16 = i_vmem[0, pl.ds(LANES * bc, LANES)]
                s16 = jnp.where(
                    t16 == 1,
                    jnp.zeros((LANES,), jnp.float32),
                    jnp.full((LANES,), SCALE, jnp.float32),
                )
                rows = iota16 + LANES * bc
                for e in range(EMB_DIM):
                    v = plsc.load_gather(
                        g_scr, [rows, jnp.full((LANES,), e, jnp.int32)]
                    )
                    o_vmem[0, e // 8, 0, e % 8, pl.ds(LANES * bc, LANES)] = (
                        v * s16
                    )

        pltpu.emit_pipeline(
            body,
            grid=(N_L, nbt),
            in_specs=[
                pl.BlockSpec((1, BB), lambda l, bt: (0, l * nbt + bt))
            ],
            out_specs=[
                pl.BlockSpec((1, 4, 1, 8, BB), lambda l, bt: (l, 0, bt, 0, 0))
            ],
            core_axis_name=("c", "s"),
            dimension_semantics=(pltpu.PARALLEL, pltpu.PARALLEL),
        )(idx_hbm, out_hbm)

    return emb_kernel


def kernel(tokens, table):
    idx = tokens.T.reshape(1, N_L * N_B).astype(jnp.int32)
    table_pad = jnp.pad(table, ((0, 0), (0, 96))).reshape(4 * 1000000, EMB_DIM)
    out5 = _build_kernel()(table_pad, idx)
    return out5.transpose(2, 4, 0, 1, 3).reshape(N_B, N_L, EMB_DIM)
